# baseline (speedup 1.0000x reference)
import functools
import math

import jax
import jax.numpy as jnp
from jax import lax
from jax.experimental import pallas as pl
from jax.experimental.pallas import tpu as pltpu
from jax.experimental.pallas import tpu_sc as plsc

EMB = 64
NUM_CORES = 2
NUM_SUBCORES = 16
NUM_WORKERS = NUM_CORES * NUM_SUBCORES
CHUNK = 128
SUB = 2
ITEM_ROWS = CHUNK * SUB
TILE_H = 8
TILE_W = 128
SCALE = math.sqrt(EMB)
LANES = 16


def _sc_embed(idx3, table, l_dim, b_dim):
    n_total = l_dim * b_dim
    per_tec = n_total // (NUM_WORKERS * ITEM_ROWS)
    items_per_l = b_dim // ITEM_ROWS
    n_b128 = b_dim // TILE_W
    mesh = plsc.VectorSubcoreMesh(core_axis_name="c", subcore_axis_name="s")

    @functools.partial(
        pl.kernel,
        mesh=mesh,
        compiler_params=pltpu.CompilerParams(
            use_tc_tiling_on_sc=False, needs_layout_passes=False),
        out_type=jax.ShapeDtypeStruct(
            (l_dim, EMB // TILE_H, n_b128, TILE_H, TILE_W), jnp.float32),
        scratch_types=[
            pltpu.VMEM((SUB * per_tec, CHUNK), jnp.int32),
            pltpu.VMEM((ITEM_ROWS, EMB), jnp.float32),
            pltpu.VMEM((ITEM_ROWS, EMB), jnp.float32),
            pltpu.VMEM((EMB // TILE_H, SUB, TILE_H, TILE_W), jnp.float32),
            pltpu.VMEM((EMB // TILE_H, SUB, TILE_H, TILE_W), jnp.float32),
            pltpu.SemaphoreType.DMA,
            pltpu.SemaphoreType.DMA,
            pltpu.SemaphoreType.DMA,
            pltpu.SemaphoreType.DMA,
        ],
    )
    def k(idx_hbm, table_hbm, out_hbm, idx_v, g0, g1, s0, s1,
          gsem0, gsem1, ssem0, ssem1):
        wid = lax.axis_index("s") * NUM_CORES + lax.axis_index("c")
        pltpu.sync_copy(idx_hbm.at[wid], idx_v)

        gbufs = ((g0, gsem0), (g1, gsem1))
        sbufs = ((s0, ssem0), (s1, ssem1))

        iota16 = lax.iota(jnp.int32, 16)
        rows = [iota16 + (t * LANES) for t in range(ITEM_ROWS // LANES)]

        def start_gathers(buf, sem, step):
            for c in range(SUB):
                src = table_hbm.at[idx_v.at[SUB * step + c]]
                dst = buf.at[pl.ds(c * CHUNK, CHUNK)]
                pltpu.make_async_copy(src, dst, sem).start()

        def wait_gathers(buf, sem):
            for c in range(SUB):
                src = table_hbm.at[idx_v.at[0]]
                dst = buf.at[pl.ds(c * CHUNK, CHUNK)]
                pltpu.make_async_copy(src, dst, sem).wait()

        def start_stores(buf, sem, step):
            m = per_tec * wid + step
            li = m // items_per_l
            c = m % items_per_l
            for e8 in range(EMB // TILE_H):
                dst = out_hbm.at[li, e8, pl.ds(SUB * c, SUB)]
                pltpu.make_async_copy(buf.at[e8], dst, sem).start()

        def wait_stores(buf, sem):
            for e8 in range(EMB // TILE_H):
                dst = out_hbm.at[0, e8, pl.ds(0, SUB)]
                pltpu.make_async_copy(buf.at[e8], dst, sem).wait()

        def tscale(src, dst):
            @pl.loop(0, TILE_H)
            def _(e8):
                @pl.loop(0, TILE_H)
                def _(er):
                    col = jnp.zeros((16,), jnp.int32) + (e8 * TILE_H + er)
                    for t in range(ITEM_ROWS // LANES):
                        v = plsc.load_gather(src, [rows[t], col])
                        tb, lane0 = divmod(t * LANES, TILE_W)
                        dst[e8, tb, er, pl.ds(lane0, LANES)] = v * SCALE

        start_gathers(g0, gsem0, 0)
        start_gathers(g1, gsem1, 1)

        for b in range(2):
            gb, gs = gbufs[b]
            sb, ss = sbufs[b]
            wait_gathers(gb, gs)
            tscale(gb, sb)
            start_stores(sb, ss, b)
            start_gathers(gb, gs, b + 2)

        @pl.loop(1, per_tec // 2 - 1)
        def _(p):
            s = 2 * p
            for b in range(2):
                gb, gs = gbufs[b]
                sb, ss = sbufs[b]
                step = s + b
                wait_gathers(gb, gs)
                wait_stores(sb, ss)
                tscale(gb, sb)
                start_stores(sb, ss, step)
                start_gathers(gb, gs, step + 2)

        for b in range(2):
            gb, gs = gbufs[b]
            sb, ss = sbufs[b]
            step = per_tec - 2 + b
            wait_gathers(gb, gs)
            wait_stores(sb, ss)
            tscale(gb, sb)
            start_stores(sb, ss, step)

        for b in range(2):
            sb, ss = sbufs[b]
            wait_stores(sb, ss)

    return k(idx3, table)


def kernel(tokens, table):
    b, l = tokens.shape
    n_total = b * l
    assert n_total % (NUM_WORKERS * ITEM_ROWS) == 0
    assert b % TILE_W == 0 and EMB % TILE_H == 0
    tok_t = tokens.T
    idx3 = tok_t.reshape(NUM_WORKERS, n_total // (NUM_WORKERS * CHUNK), CHUNK)
    out5 = _sc_embed(idx3, table, l, b)
    return out5.transpose(2, 4, 0, 1, 3).reshape(b, l, EMB)
